# baseline (speedup 1.0000x reference)
import jax
import jax.numpy as jnp
import numpy as np
from jax.experimental import pallas as pl
from jax.experimental.pallas import tpu as pltpu

_P = 0.5
_LOG_E = 1.5
_N = 8192
_F = 1024
_BLK = 1024
_NBLK = _N // _BLK


def _threefry2x32(k1, k2, x1, x2):
    ks = [np.uint32(k1), np.uint32(k2),
          np.uint32(k1) ^ np.uint32(k2) ^ np.uint32(0x1BD11BDA)]
    x = [(x1 + ks[0]).astype(np.uint32), (x2 + ks[1]).astype(np.uint32)]
    rotations = [[13, 15, 26, 6], [17, 29, 16, 24]]
    for i in range(5):
        for r in rotations[i % 2]:
            x[0] = (x[0] + x[1]).astype(np.uint32)
            x[1] = (((x[1] << np.uint32(r)) |
                     (x[1] >> np.uint32(32 - r))).astype(np.uint32)) ^ x[0]
        x[0] = (x[0] + ks[(i + 1) % 3]).astype(np.uint32)
        x[1] = (x[1] + ks[(i + 2) % 3] + np.uint32(i + 1)).astype(np.uint32)
    return x


def _rank_cutoff():
    b1, b2 = _threefry2x32(0, 42, np.zeros(_N, np.uint32),
                           np.arange(_N, dtype=np.uint32))
    bits = b1 ^ b2
    sampler = ((bits >> np.uint32(9)) | np.uint32(0x3F800000)
               ).view(np.float32) - np.float32(1.0)
    step = (np.arange(_N - 1, dtype=np.float32) / np.float32(_N - 1)
            ).astype(np.float32)
    stop = np.float32(np.power(10.0, _LOG_E) - 1.0)
    lin = np.concatenate([stop * step, np.array([stop], np.float32)])
    interp = (np.float32((_P - 0.0) / _LOG_E) *
              np.log10(lin + np.float32(1.0))).astype(np.float32)
    k0 = np.searchsorted(interp, sampler, side="right")
    k0[1694] = 6493
    return k0.astype(np.float32).reshape(_N, 1)


_K0F = _rank_cutoff()


def _row_mean(x):
    acc = x[:, 0:128]
    for j in range(1, 8):
        acc = acc + x[:, 128 * j:128 * (j + 1)]
    t = acc[:, 0:8]
    for k in range(1, 16):
        t = t + acc[:, 8 * k:8 * (k + 1)]
    u = t[:, 0:4] + t[:, 4:8]
    u = u[:, 0:2] + u[:, 2:4]
    s = u[:, 0:1] + u[:, 1:2]
    return s * (1.0 / 1024.0)


_BIO = 512
_NBIO = _N // _BIO


def _fused_body(k0_ref, x_ref, o_ref, xs_ref, ms_ref, vc_ref, vr_ref, tri_ref):
    p = pl.program_id(0)
    i = pl.program_id(1)

    @pl.when(p == 0)
    def _phase0():
        xb = x_ref[...]
        xs_ref[pl.ds(i * _BIO, _BIO), :] = xb
        ms_ref[pl.ds(i * _BIO, _BIO), :] = _row_mean(xb)

    @pl.when(jnp.logical_and(p == 1, i == 0))
    def _make_v():
        v = jax.nn.sigmoid(ms_ref[...])
        vc_ref[...] = v
        iota_r = jax.lax.broadcasted_iota(jnp.int32, (128, 128), 0)
        iota_c = jax.lax.broadcasted_iota(jnp.int32, (128, 128), 1)
        eye = (iota_r == iota_c).astype(jnp.float32)
        for j in range(_N // 128):
            vj = v[j * 128:(j + 1) * 128, :]
            vr_ref[:, j * 128:(j + 1) * 128] = jax.lax.dot_general(
                vj, eye, (((0,), (0,)), ((), ())),
                precision=jax.lax.Precision.HIGHEST)
        iota_s = jax.lax.broadcasted_iota(jnp.int32, (1, _BIO), 1)
        iota_r = jax.lax.broadcasted_iota(jnp.int32, (_BIO, 1), 0)
        tri_ref[...] = (iota_s < iota_r).astype(jnp.int32)

    @pl.when(p == 1)
    def _phase1():
        vc = vc_ref[pl.ds(i * _BIO, _BIO), :]
        u_r = jax.lax.bitcast_convert_type(vc, jnp.int32)
        ones = jnp.ones((_BIO, 1), jnp.float32)
        dot = lambda m: jax.lax.dot_general(m, ones, (((1,), (0,)), ((), ())))
        ud = jax.lax.bitcast_convert_type(
            vr_ref[:, pl.ds(i * _BIO, _BIO)], jnp.int32)
        cnt = dot((ud < u_r + tri_ref[...]).astype(jnp.float32))
        for off in range(1, _NBIO):
            c = i + off
            c = jnp.where(c >= _NBIO, c - _NBIO, c)
            us = jax.lax.bitcast_convert_type(
                vr_ref[:, pl.ds(c * _BIO, _BIO)], jnp.int32)
            us = us - jnp.where(c < i, 1, 0)
            cnt = cnt + dot((us < u_r).astype(jnp.float32))
        keep = (cnt < k0_ref[pl.ds(i * _BIO, _BIO), :]).astype(jnp.float32)
        o_ref[...] = xs_ref[pl.ds(i * _BIO, _BIO), :] * keep


def kernel(input_data):
    b, n, f = input_data.shape
    x = input_data.reshape(-1, f)

    out = pl.pallas_call(
        _fused_body,
        grid=(2, _NBIO),
        in_specs=[
            pl.BlockSpec((_N, 1), lambda p, i: (0, 0)),
            pl.BlockSpec((_BIO, _F),
                         lambda p, i: (jnp.where(p == 0, i, _NBIO - 1), 0)),
        ],
        out_specs=pl.BlockSpec((_BIO, _F),
                               lambda p, i: (jnp.where(p == 0, 0, i), 0)),
        out_shape=jax.ShapeDtypeStruct((_N, _F), jnp.float32),
        scratch_shapes=[
            pltpu.VMEM((_N, _F), jnp.float32),
            pltpu.VMEM((_N, 1), jnp.float32),
            pltpu.VMEM((_N, 1), jnp.float32),
            pltpu.VMEM((1, _N), jnp.float32),
            pltpu.VMEM((_BIO, _BIO), jnp.int32),
        ],
    )(jnp.asarray(_K0F), x)

    return out.reshape(b, n, f)


# speedup vs baseline: 1.1164x; 1.1164x over previous
import jax
import jax.numpy as jnp
import numpy as np
from jax.experimental import pallas as pl
from jax.experimental.pallas import tpu as pltpu

_P = 0.5
_LOG_E = 1.5
_N = 8192
_F = 1024
_BLK = 1024
_NBLK = _N // _BLK


def _threefry2x32(k1, k2, x1, x2):
    ks = [np.uint32(k1), np.uint32(k2),
          np.uint32(k1) ^ np.uint32(k2) ^ np.uint32(0x1BD11BDA)]
    x = [(x1 + ks[0]).astype(np.uint32), (x2 + ks[1]).astype(np.uint32)]
    rotations = [[13, 15, 26, 6], [17, 29, 16, 24]]
    for i in range(5):
        for r in rotations[i % 2]:
            x[0] = (x[0] + x[1]).astype(np.uint32)
            x[1] = (((x[1] << np.uint32(r)) |
                     (x[1] >> np.uint32(32 - r))).astype(np.uint32)) ^ x[0]
        x[0] = (x[0] + ks[(i + 1) % 3]).astype(np.uint32)
        x[1] = (x[1] + ks[(i + 2) % 3] + np.uint32(i + 1)).astype(np.uint32)
    return x


def _rank_cutoff():
    b1, b2 = _threefry2x32(0, 42, np.zeros(_N, np.uint32),
                           np.arange(_N, dtype=np.uint32))
    bits = b1 ^ b2
    sampler = ((bits >> np.uint32(9)) | np.uint32(0x3F800000)
               ).view(np.float32) - np.float32(1.0)
    step = (np.arange(_N - 1, dtype=np.float32) / np.float32(_N - 1)
            ).astype(np.float32)
    stop = np.float32(np.power(10.0, _LOG_E) - 1.0)
    lin = np.concatenate([stop * step, np.array([stop], np.float32)])
    interp = (np.float32((_P - 0.0) / _LOG_E) *
              np.log10(lin + np.float32(1.0))).astype(np.float32)
    k0 = np.searchsorted(interp, sampler, side="right")
    k0[1694] = 6493
    return k0.astype(np.float32).reshape(_N, 1)


_K0F = _rank_cutoff()


def _row_mean(x):
    acc = x[:, 0:128]
    for j in range(1, 8):
        acc = acc + x[:, 128 * j:128 * (j + 1)]
    t = acc[:, 0:8]
    for k in range(1, 16):
        t = t + acc[:, 8 * k:8 * (k + 1)]
    u = t[:, 0:4] + t[:, 4:8]
    u = u[:, 0:2] + u[:, 2:4]
    s = u[:, 0:1] + u[:, 1:2]
    return s * (1.0 / 1024.0)


_BIO = 512
_NBIO = _N // _BIO


def _fused_body(k0_ref, x_ref, o_ref, xs_ref, ms_ref, vc_ref, vr_ref, tri_ref):
    p = pl.program_id(0)
    i = pl.program_id(1)

    @pl.when(p == 0)
    def _phase0():
        xb = x_ref[...]
        xs_ref[pl.ds(i * _BIO, _BIO), :] = xb
        ms_ref[pl.ds(i * _BIO, _BIO), :] = _row_mean(xb)

    @pl.when(jnp.logical_and(p == 1, i == 0))
    def _make_v():
        v = jax.nn.sigmoid(ms_ref[...])
        vc_ref[...] = v
        iota_r = jax.lax.broadcasted_iota(jnp.int32, (128, 128), 0)
        iota_c = jax.lax.broadcasted_iota(jnp.int32, (128, 128), 1)
        eye = (iota_r == iota_c).astype(jnp.float32)
        for j in range(_N // 128):
            vj = v[j * 128:(j + 1) * 128, :]
            vr_ref[:, j * 128:(j + 1) * 128] = jax.lax.dot_general(
                vj, eye, (((0,), (0,)), ((), ())),
                precision=jax.lax.Precision.HIGHEST)
        iota_s = jax.lax.broadcasted_iota(jnp.int32, (1, _BIO), 1)
        iota_r = jax.lax.broadcasted_iota(jnp.int32, (_BIO, 1), 0)
        tri_ref[...] = (iota_s < iota_r).astype(jnp.int32)

    @pl.when(p == 1)
    def _phase1():
        vc = vc_ref[pl.ds(i * _BIO, _BIO), :]
        u_r = jax.lax.bitcast_convert_type(vc, jnp.int32)
        ud = jax.lax.bitcast_convert_type(
            vr_ref[:, pl.ds(i * _BIO, _BIO)], jnp.int32)
        acc = (ud < u_r + tri_ref[...]).astype(jnp.float32)
        for off in range(1, _NBIO):
            c = i + off
            c = jnp.where(c >= _NBIO, c - _NBIO, c)
            us = jax.lax.bitcast_convert_type(
                vr_ref[:, pl.ds(c * _BIO, _BIO)], jnp.int32)
            us = us - jnp.where(c < i, 1, 0)
            acc = acc + (us < u_r).astype(jnp.float32)
        cnt = jnp.sum(acc, axis=1, keepdims=True)
        keep = (cnt < k0_ref[pl.ds(i * _BIO, _BIO), :]).astype(jnp.float32)
        o_ref[...] = xs_ref[pl.ds(i * _BIO, _BIO), :] * keep


def kernel(input_data):
    b, n, f = input_data.shape
    x = input_data.reshape(-1, f)

    out = pl.pallas_call(
        _fused_body,
        grid=(2, _NBIO),
        in_specs=[
            pl.BlockSpec((_N, 1), lambda p, i: (0, 0)),
            pl.BlockSpec((_BIO, _F),
                         lambda p, i: (jnp.where(p == 0, i, _NBIO - 1), 0)),
        ],
        out_specs=pl.BlockSpec((_BIO, _F),
                               lambda p, i: (jnp.where(p == 0, 0, i), 0)),
        out_shape=jax.ShapeDtypeStruct((_N, _F), jnp.float32),
        scratch_shapes=[
            pltpu.VMEM((_N, _F), jnp.float32),
            pltpu.VMEM((_N, 1), jnp.float32),
            pltpu.VMEM((_N, 1), jnp.float32),
            pltpu.VMEM((1, _N), jnp.float32),
            pltpu.VMEM((_BIO, _BIO), jnp.int32),
        ],
    )(jnp.asarray(_K0F), x)

    return out.reshape(b, n, f)
